# baseline (speedup 1.0000x reference)
import functools

import jax
import jax.numpy as jnp
from jax import lax
from jax.experimental import pallas as pl
from jax.experimental.pallas import tpu as pltpu
from jax.experimental.pallas import tpu_sc as plsc

_B = 4
_N = 4096
_NW = 32
_ROWS_PER_W = 2 * _B * _N // _NW
_KBLK = 8
_NBLK = _ROWS_PER_W // (16 * _KBLK)


def _sc_body(rowsf_hbm, rowsb_hbm, othf_hbm, othb_hbm, out_hbm,
             rf_v, rb_v, qf_v, qb_v, min_v):
    d = lax.axis_index("c")
    s = lax.axis_index("s")
    b = s // 4
    q = s % 4

    pltpu.sync_copy(rowsf_hbm.at[d, b], rf_v)
    pltpu.sync_copy(rowsb_hbm.at[d, b], rb_v)
    pltpu.sync_copy(othf_hbm.at[d, b], qf_v)
    pltpu.sync_copy(othb_hbm.at[d, b], qb_v)
    row0 = q * _ROWS_PER_W

    for kb in range(_NBLK):
        base = row0 + kb * 16 * _KBLK
        rxb = [rb_v[0, pl.ds(base + j * 16, 16)] for j in range(_KBLK)]
        ryb = [rb_v[1, pl.ds(base + j * 16, 16)] for j in range(_KBLK)]
        rzb = [rb_v[2, pl.ds(base + j * 16, 16)] for j in range(_KBLK)]
        rn = []
        for j in range(_KBLK):
            rfx = rf_v[0, pl.ds(base + j * 16, 16)]
            rfy = rf_v[1, pl.ds(base + j * 16, 16)]
            rfz = rf_v[2, pl.ds(base + j * 16, 16)]
            rn.append(rfx * rfx + rfy * rfy + rfz * rfz)
        inf = jnp.full((16,), jnp.float32(jnp.inf), jnp.float32)

        def mstep(mc, mins, rxb=rxb, ryb=ryb, rzb=rzb, rn=rn):
            qxv = qb_v[0, pl.ds(mc * 16, 16)]
            qyv = qb_v[1, pl.ds(mc * 16, 16)]
            qzv = qb_v[2, pl.ds(mc * 16, 16)]
            fxv = qf_v[0, pl.ds(mc * 16, 16)]
            fyv = qf_v[1, pl.ds(mc * 16, 16)]
            fzv = qf_v[2, pl.ds(mc * 16, 16)]
            qnv = fxv * fxv + fyv * fyv + fzv * fzv
            cur = list(mins)
            for i in range(16):
                qx = qxv[i]
                qy = qyv[i]
                qz = qzv[i]
                qn = qnv[i]
                for j in range(_KBLK):
                    t = rxb[j] * qx + ryb[j] * qy + rzb[j] * qz
                    dd = (rn[j] + qn) - 2.0 * t
                    cur[j] = jnp.minimum(cur[j], dd)
            return tuple(cur)

        mins = lax.fori_loop(0, _N // 16, mstep, tuple(inf for _ in range(_KBLK)))
        for j in range(_KBLK):
            min_v[pl.ds(kb * 16 * _KBLK + j * 16, 16)] = jnp.maximum(
                mins[j], jnp.float32(0.0))

    pltpu.sync_copy(min_v, out_hbm.at[d, b, pl.ds(q * _ROWS_PER_W, _ROWS_PER_W)])


_sc_minsq = functools.partial(
    pl.kernel,
    out_type=jax.ShapeDtypeStruct((2, _B, _N), jnp.float32),
    mesh=plsc.VectorSubcoreMesh(core_axis_name="c", subcore_axis_name="s"),
    scratch_types=[
        pltpu.VMEM((3, _N), jnp.float32),
        pltpu.VMEM((3, _N), jnp.float32),
        pltpu.VMEM((3, _N), jnp.float32),
        pltpu.VMEM((3, _N), jnp.float32),
        pltpu.VMEM((_ROWS_PER_W,), jnp.float32),
    ],
)(_sc_body)


def _tc_reduce_body(x_ref, o_ref):
    o_ref[0, 0] = jnp.sum(jnp.sqrt(x_ref[...])) * (1.0 / (2 * _B * _N))


def kernel(array1, array2):
    a1t = jnp.transpose(array1, (0, 2, 1))
    a2t = jnp.transpose(array2, (0, 2, 1))
    a1b16, a2b16 = lax.optimization_barrier(
        (a1t.astype(jnp.bfloat16), a2t.astype(jnp.bfloat16)))
    a1b = a1b16.astype(jnp.float32)
    a2b = a2b16.astype(jnp.float32)
    rows_f = jnp.stack([a1t, a2t])
    rows_b = jnp.stack([a1b, a2b])
    oth_f = jnp.stack([a2t, a1t])
    oth_b = jnp.stack([a2b, a1b])
    minsq = _sc_minsq(rows_f, rows_b, oth_f, oth_b)
    flat = minsq.reshape(2 * _B * _N // 128, 128)
    out = pl.pallas_call(
        _tc_reduce_body,
        out_shape=jax.ShapeDtypeStruct((1, 1), jnp.float32),
        out_specs=pl.BlockSpec(memory_space=pltpu.SMEM),
    )(flat)
    return out[0, 0]


# speedup vs baseline: 1.0995x; 1.0995x over previous
import functools

import jax
import jax.numpy as jnp
from jax import lax
from jax.experimental import pallas as pl
from jax.experimental.pallas import tpu as pltpu
from jax.experimental.pallas import tpu_sc as plsc

_B = 4
_N = 4096
_NW = 32
_ROWS_PER_W = 2 * _B * _N // _NW
_KBLK = 8
_NBLK = _ROWS_PER_W // (16 * _KBLK)


def _sc_body(rowsf_hbm, rowsb_hbm, othf_hbm, othb_hbm, out_hbm,
             rf_v, rb_v, qf_v, qb_v, min_v):
    d = lax.axis_index("c")
    s = lax.axis_index("s")
    b = s // 4
    q = s % 4

    pltpu.sync_copy(rowsf_hbm.at[d, b], rf_v)
    pltpu.sync_copy(rowsb_hbm.at[d, b], rb_v)
    pltpu.sync_copy(othf_hbm.at[d, b], qf_v)
    pltpu.sync_copy(othb_hbm.at[d, b], qb_v)
    row0 = q * _ROWS_PER_W

    for kb in range(_NBLK):
        base = row0 + kb * 16 * _KBLK
        rxb = [rb_v[0, pl.ds(base + j * 16, 16)] for j in range(_KBLK)]
        ryb = [rb_v[1, pl.ds(base + j * 16, 16)] for j in range(_KBLK)]
        rzb = [rb_v[2, pl.ds(base + j * 16, 16)] for j in range(_KBLK)]
        rn = []
        for j in range(_KBLK):
            rfx = rf_v[0, pl.ds(base + j * 16, 16)]
            rfy = rf_v[1, pl.ds(base + j * 16, 16)]
            rfz = rf_v[2, pl.ds(base + j * 16, 16)]
            rn.append(rfx * rfx + rfy * rfy + rfz * rfz)
        inf = jnp.full((16,), jnp.float32(jnp.inf), jnp.float32)

        def mstep(mc, mins, rxb=rxb, ryb=ryb, rzb=rzb, rn=rn):
            qxv = qb_v[0, pl.ds(mc * 16, 16)]
            qyv = qb_v[1, pl.ds(mc * 16, 16)]
            qzv = qb_v[2, pl.ds(mc * 16, 16)]
            fxv = qf_v[0, pl.ds(mc * 16, 16)]
            fyv = qf_v[1, pl.ds(mc * 16, 16)]
            fzv = qf_v[2, pl.ds(mc * 16, 16)]
            qnv = fxv * fxv + fyv * fyv + fzv * fzv
            cur = list(mins)
            for i in range(16):
                qx = qxv[i]
                qy = qyv[i]
                qz = qzv[i]
                qn = qnv[i]
                for j in range(_KBLK):
                    t = rxb[j] * qx + ryb[j] * qy + rzb[j] * qz
                    dd = qn - 2.0 * t
                    cur[j] = jnp.minimum(cur[j], dd)
            return tuple(cur)

        mins = lax.fori_loop(0, _N // 16, mstep, tuple(inf for _ in range(_KBLK)))
        for j in range(_KBLK):
            min_v[pl.ds(kb * 16 * _KBLK + j * 16, 16)] = jnp.maximum(
                rn[j] + mins[j], jnp.float32(0.0))

    pltpu.sync_copy(min_v, out_hbm.at[d, b, pl.ds(q * _ROWS_PER_W, _ROWS_PER_W)])


_sc_minsq = functools.partial(
    pl.kernel,
    out_type=jax.ShapeDtypeStruct((2, _B, _N), jnp.float32),
    mesh=plsc.VectorSubcoreMesh(core_axis_name="c", subcore_axis_name="s"),
    scratch_types=[
        pltpu.VMEM((3, _N), jnp.float32),
        pltpu.VMEM((3, _N), jnp.float32),
        pltpu.VMEM((3, _N), jnp.float32),
        pltpu.VMEM((3, _N), jnp.float32),
        pltpu.VMEM((_ROWS_PER_W,), jnp.float32),
    ],
)(_sc_body)


def _tc_reduce_body(x_ref, o_ref):
    o_ref[0, 0] = jnp.sum(jnp.sqrt(x_ref[...])) * (1.0 / (2 * _B * _N))


def kernel(array1, array2):
    a1t = jnp.transpose(array1, (0, 2, 1))
    a2t = jnp.transpose(array2, (0, 2, 1))
    a1b16, a2b16 = lax.optimization_barrier(
        (a1t.astype(jnp.bfloat16), a2t.astype(jnp.bfloat16)))
    a1b = a1b16.astype(jnp.float32)
    a2b = a2b16.astype(jnp.float32)
    rows_f = jnp.stack([a1t, a2t])
    rows_b = jnp.stack([a1b, a2b])
    oth_f = jnp.stack([a2t, a1t])
    oth_b = jnp.stack([a2b, a1b])
    minsq = _sc_minsq(rows_f, rows_b, oth_f, oth_b)
    flat = minsq.reshape(2 * _B * _N // 128, 128)
    out = pl.pallas_call(
        _tc_reduce_body,
        out_shape=jax.ShapeDtypeStruct((1, 1), jnp.float32),
        out_specs=pl.BlockSpec(memory_space=pltpu.SMEM),
    )(flat)
    return out[0, 0]


# speedup vs baseline: 3.0992x; 2.8187x over previous
import functools

import jax
import jax.numpy as jnp
from jax import lax
from jax.experimental import pallas as pl
from jax.experimental.pallas import tpu as pltpu
from jax.experimental.pallas import tpu_sc as plsc

_B = 4
_N = 4096
_S = 512
_T = _N - _S
_RT = 512
_NT = _T // _RT
_ND = 2 * _B
_WPP = 4
_RPW = _S // _WPP
_KBLK = 8
_NBLK = _RPW // (16 * _KBLK)


def _sc_body(rowsb_hbm, rowsf_hbm, othb_hbm, othf_hbm, out_hbm,
             rb_v, rf_v, qb_v, qf_v, min_v):
    d = lax.axis_index("c")
    s = lax.axis_index("s")
    b = s // _WPP
    q = s % _WPP

    pltpu.sync_copy(rowsb_hbm.at[d, b], rb_v)
    pltpu.sync_copy(rowsf_hbm.at[d, b], rf_v)
    pltpu.sync_copy(othb_hbm.at[d, b], qb_v)
    pltpu.sync_copy(othf_hbm.at[d, b], qf_v)
    row0 = _T + q * _RPW

    for kb in range(_NBLK):
        base = row0 + kb * 16 * _KBLK
        rxb = [rb_v[0, pl.ds(base + j * 16, 16)] for j in range(_KBLK)]
        ryb = [rb_v[1, pl.ds(base + j * 16, 16)] for j in range(_KBLK)]
        rzb = [rb_v[2, pl.ds(base + j * 16, 16)] for j in range(_KBLK)]
        rn = []
        for j in range(_KBLK):
            rfx = rf_v[0, pl.ds(base + j * 16, 16)]
            rfy = rf_v[1, pl.ds(base + j * 16, 16)]
            rfz = rf_v[2, pl.ds(base + j * 16, 16)]
            rn.append(rfx * rfx + rfy * rfy + rfz * rfz)
        inf = jnp.full((16,), jnp.float32(jnp.inf), jnp.float32)

        def mstep(mc, mins, rxb=rxb, ryb=ryb, rzb=rzb):
            qxv = qb_v[0, pl.ds(mc * 16, 16)]
            qyv = qb_v[1, pl.ds(mc * 16, 16)]
            qzv = qb_v[2, pl.ds(mc * 16, 16)]
            fxv = qf_v[0, pl.ds(mc * 16, 16)]
            fyv = qf_v[1, pl.ds(mc * 16, 16)]
            fzv = qf_v[2, pl.ds(mc * 16, 16)]
            qnv = fxv * fxv + fyv * fyv + fzv * fzv
            cur = list(mins)
            for i in range(16):
                qx = qxv[i]
                qy = qyv[i]
                qz = qzv[i]
                qn = qnv[i]
                for j in range(_KBLK):
                    t = rxb[j] * qx + ryb[j] * qy + rzb[j] * qz
                    dd = qn - 2.0 * t
                    cur[j] = jnp.minimum(cur[j], dd)
            return tuple(cur)

        mins = lax.fori_loop(0, _N // 16, mstep, tuple(inf for _ in range(_KBLK)))
        for j in range(_KBLK):
            min_v[pl.ds(kb * 16 * _KBLK + j * 16, 16)] = jnp.maximum(
                rn[j] + mins[j], jnp.float32(0.0))

    pltpu.sync_copy(min_v, out_hbm.at[d, b, pl.ds(q * _RPW, _RPW)])


_sc_minsq = functools.partial(
    pl.kernel,
    out_type=jax.ShapeDtypeStruct((2, _B, _S), jnp.float32),
    mesh=plsc.VectorSubcoreMesh(core_axis_name="c", subcore_axis_name="s"),
    scratch_types=[
        pltpu.VMEM((3, _N), jnp.float32),
        pltpu.VMEM((3, _N), jnp.float32),
        pltpu.VMEM((3, _N), jnp.float32),
        pltpu.VMEM((3, _N), jnp.float32),
        pltpu.VMEM((_RPW,), jnp.float32),
    ],
)(_sc_body)


def _tc_nn_body(lb_ref, lf_ref, rb_ref, rf_ref, o_ref):
    lb = lb_ref[0]
    rb = rb_ref[0]
    inner = lax.dot_general(lb, rb, (((1,), (0,)), ((), ())),
                            preferred_element_type=jnp.float32)
    rf = rf_ref[0]
    qn = jnp.sum(rf * rf, axis=0)
    rowmin = jnp.min(inner + qn[None, :], axis=1)
    lf = lf_ref[0]
    rn = jnp.sum(lf * lf, axis=1)
    o_ref[0, 0, :] = jnp.maximum(rowmin + rn, 0.0)


def _tc_minsq(rows_b2, rows_f, oth_b, oth_f):
    return pl.pallas_call(
        _tc_nn_body,
        grid=(_ND, _NT),
        in_specs=[
            pl.BlockSpec((1, _RT, 8), lambda g, t: (g, t, 0)),
            pl.BlockSpec((1, _RT, 8), lambda g, t: (g, t, 0)),
            pl.BlockSpec((1, 8, _N), lambda g, t: (g, 0, 0)),
            pl.BlockSpec((1, 8, _N), lambda g, t: (g, 0, 0)),
        ],
        out_specs=pl.BlockSpec((1, 1, _RT), lambda g, t: (g * _NT + t, 0, 0)),
        out_shape=jax.ShapeDtypeStruct((_ND * _NT, 1, _RT), jnp.float32),
    )(rows_b2, rows_f, oth_b, oth_f)


def _tc_reduce_body(x_ref, o_ref):
    o_ref[0, 0] = jnp.sum(jnp.sqrt(x_ref[...])) * (1.0 / (2 * _B * _N))


def kernel(array1, array2):
    a1t = jnp.transpose(array1, (0, 2, 1))
    a2t = jnp.transpose(array2, (0, 2, 1))
    a1b16, a2b16 = lax.optimization_barrier(
        (a1t.astype(jnp.bfloat16), a2t.astype(jnp.bfloat16)))
    a1b = a1b16.astype(jnp.float32)
    a2b = a2b16.astype(jnp.float32)

    rows_f = jnp.stack([a1t, a2t])
    rows_b = jnp.stack([a1b, a2b])
    oth_f = jnp.stack([a2t, a1t])
    oth_b = jnp.stack([a2b, a1b])

    minsq_sc = _sc_minsq(rows_b, rows_f, oth_b, oth_f)

    pad = jnp.zeros((2, _B, 5, _N), jnp.float32)
    rows_f8 = jnp.concatenate([rows_f, pad], axis=2)
    rows_b8 = jnp.concatenate([rows_b, pad], axis=2)
    oth_f8 = jnp.concatenate([oth_f, pad], axis=2)
    oth_b8 = jnp.concatenate([oth_b, pad], axis=2)
    lhs_b2 = (-2.0 * rows_b8).reshape(_ND, 8, _N).transpose(0, 2, 1)
    lhs_f = rows_f8.reshape(_ND, 8, _N).transpose(0, 2, 1)
    rhs_b = oth_b8.reshape(_ND, 8, _N)
    rhs_f = oth_f8.reshape(_ND, 8, _N)
    minsq_tc = _tc_minsq(lhs_b2[:, :_T], lhs_f[:, :_T], rhs_b, rhs_f)
    minsq_tc = minsq_tc.reshape(2, _B, _T)

    minsq = jnp.concatenate([minsq_tc, minsq_sc], axis=-1)
    flat = minsq.reshape(2 * _B * _N // 128, 128)
    out = pl.pallas_call(
        _tc_reduce_body,
        out_shape=jax.ShapeDtypeStruct((1, 1), jnp.float32),
        out_specs=pl.BlockSpec(memory_space=pltpu.SMEM),
    )(flat)
    return out[0, 0]
